# initial kernel scaffold (unmeasured)
import jax
import jax.numpy as jnp
from jax import lax
from jax.experimental import pallas as pl
from jax.experimental.pallas import tpu as pltpu

N_DEV = 8
SQ = 1024
SKV = 1024
D = 1024
HQ = 8
DH = 128
SCALE = 0.08838834764831843


def kernel(x, Wq, K_ext, V_ext, Wo):
    xb = x[0].astype(jnp.bfloat16)
    wq = Wq.astype(jnp.bfloat16)
    wo = Wo.astype(jnp.bfloat16)

    def body(xb_ref, wq_ref, wo_ref, kext_ref, vext_ref, out_ref,
             buf, qbuf, contrib, bias, ktile, vtile,
             send_sems, recv_sems, kv_sems, credit_sem):
        my = lax.axis_index("i")
        left = lax.rem(my + N_DEV - 1, N_DEV)
        right = lax.rem(my + 1, N_DEV)

        barrier = pltpu.get_barrier_semaphore()
        for nbr in (left, right):
            pl.semaphore_signal(barrier, inc=1, device_id=(nbr,),
                                device_id_type=pl.DeviceIdType.MESH)
        pl.semaphore_wait(barrier, 2)

        rblk = lax.broadcasted_iota(jnp.int32, (SQ, SKV), 0) // 64
        cblk = lax.broadcasted_iota(jnp.int32, (SQ, SKV), 1) // 64
        bias[...] = jnp.where(cblk <= rblk, 0.0, -1e9).astype(jnp.bfloat16)

        def compute_contribution(b, x_src):
            q = jnp.dot(x_src, wq_ref[...],
                        preferred_element_type=jnp.float32)
            qbuf[...] = (q * SCALE).astype(jnp.bfloat16)
            contrib[...] = jnp.zeros((SQ, D), jnp.float32)

            def head_body(h, carry):
                gh = my * HQ + h
                ck = pltpu.make_async_copy(
                    kext_ref.at[b, :, gh, :], ktile, kv_sems.at[0])
                cv = pltpu.make_async_copy(
                    vext_ref.at[b, :, gh, :], vtile, kv_sems.at[1])
                ck.start()
                cv.start()
                qh = qbuf[:, pl.ds(h * DH, DH)]
                ck.wait()
                kk = ktile[...].astype(jnp.bfloat16)
                s = lax.dot_general(
                    qh, kk, (((1,), (1,)), ((), ())),
                    preferred_element_type=jnp.float32)
                s = s + bias[...]
                m = jnp.max(s, axis=-1, keepdims=True)
                e = jnp.exp(s - m)
                den = jnp.sum(e, axis=-1, keepdims=True)
                p = (e / den).astype(jnp.bfloat16)
                cv.wait()
                vv = vtile[...].astype(jnp.bfloat16)
                ctx = jnp.dot(p, vv,
                              preferred_element_type=jnp.float32)
                ctx = ctx.astype(jnp.bfloat16)
                woh = wo_ref[pl.ds(h * DH, DH), :]
                contrib[...] += jnp.dot(ctx, woh,
                                        preferred_element_type=jnp.float32)
                return carry

            lax.fori_loop(0, HQ, head_body, 0)

        buf[0, 0] = xb_ref[...]
        compute_contribution(my, xb_ref[...])
        buf[0, 1] = contrib[...].astype(jnp.bfloat16)

        for s in range(1, N_DEV + 1):
            src_slot = (s - 1) % 2
            dst_slot = s % 2
            if s >= 2:
                pl.semaphore_wait(credit_sem, 1)
            rdma = pltpu.make_async_remote_copy(
                src_ref=buf.at[src_slot],
                dst_ref=buf.at[dst_slot],
                send_sem=send_sems.at[src_slot],
                recv_sem=recv_sems.at[dst_slot],
                device_id=(right,),
                device_id_type=pl.DeviceIdType.MESH,
            )
            rdma.start()
            rdma.wait()
            if s <= N_DEV - 1:
                pl.semaphore_signal(credit_sem, inc=1, device_id=(left,),
                                    device_id_type=pl.DeviceIdType.MESH)
                b = lax.rem(my - s + N_DEV, N_DEV)
                compute_contribution(b, buf[dst_slot, 0])
                buf[dst_slot, 1] = (
                    buf[dst_slot, 1].astype(jnp.float32) + contrib[...]
                ).astype(jnp.bfloat16)
            else:
                out_ref[0] = buf[dst_slot, 1].astype(jnp.float32)

    out = pl.pallas_call(
        body,
        out_shape=jax.ShapeDtypeStruct((1, SQ, D), jnp.float32),
        in_specs=[
            pl.BlockSpec(memory_space=pltpu.VMEM),
            pl.BlockSpec(memory_space=pltpu.VMEM),
            pl.BlockSpec(memory_space=pltpu.VMEM),
            pl.BlockSpec(memory_space=pltpu.ANY),
            pl.BlockSpec(memory_space=pltpu.ANY),
        ],
        out_specs=pl.BlockSpec(memory_space=pltpu.VMEM),
        scratch_shapes=[
            pltpu.VMEM((2, 2, SQ, D), jnp.bfloat16),
            pltpu.VMEM((SQ, D), jnp.bfloat16),
            pltpu.VMEM((SQ, D), jnp.float32),
            pltpu.VMEM((SQ, SKV), jnp.bfloat16),
            pltpu.VMEM((SKV, DH), jnp.float32),
            pltpu.VMEM((SKV, DH), jnp.float32),
            pltpu.SemaphoreType.DMA((2,)),
            pltpu.SemaphoreType.DMA((2,)),
            pltpu.SemaphoreType.DMA((2,)),
            pltpu.SemaphoreType.REGULAR,
        ],
        compiler_params=pltpu.CompilerParams(collective_id=0),
    )(xb, wq, wo, K_ext, V_ext)
    return out


# baseline (device time: 735199 ns/iter reference)
import jax
import jax.numpy as jnp
from jax import lax
from jax.experimental import pallas as pl
from jax.experimental.pallas import tpu as pltpu

N_DEV = 8
SQ = 1024
SKV = 1024
D = 1024
HQ = 8
DH = 128
SCALE = 0.08838834764831843


def kernel(x, Wq, K_ext, V_ext, Wo):
    xb = x[0].astype(jnp.bfloat16)
    wq = Wq.astype(jnp.bfloat16)
    wo = Wo.astype(jnp.bfloat16)

    def body(xb_ref, wq_ref, wo_ref, kext_ref, vext_ref, out_ref,
             buf, qbuf, contrib, bias, ktile, vtile,
             send_sems, recv_sems, kv_sems, credit_sem):
        my = lax.axis_index("i")
        left = lax.rem(my + N_DEV - 1, N_DEV)
        right = lax.rem(my + 1, N_DEV)

        barrier = pltpu.get_barrier_semaphore()
        for nbr in (left, right):
            pl.semaphore_signal(barrier, inc=1, device_id=(nbr,),
                                device_id_type=pl.DeviceIdType.MESH)
        pl.semaphore_wait(barrier, 2)

        rblk = lax.broadcasted_iota(jnp.int32, (SQ, SKV), 0) // 64
        cblk = lax.broadcasted_iota(jnp.int32, (SQ, SKV), 1) // 64
        bias[...] = jnp.where(cblk <= rblk, 0.0, -1e9).astype(jnp.bfloat16)

        def compute_contribution(b, x_src):
            q = jnp.dot(x_src, wq_ref[...],
                        preferred_element_type=jnp.float32)
            qbuf[...] = (q * SCALE).astype(jnp.bfloat16)
            contrib[...] = jnp.zeros((SQ, D), jnp.float32)

            def head_body(h, carry):
                gh = my * HQ + h
                ck = pltpu.make_async_copy(
                    kext_ref.at[b, :, gh, :], ktile, kv_sems.at[0])
                cv = pltpu.make_async_copy(
                    vext_ref.at[b, :, gh, :], vtile, kv_sems.at[1])
                ck.start()
                cv.start()
                qh = qbuf[:, pl.ds(h * DH, DH)]
                ck.wait()
                kk = ktile[...].astype(jnp.bfloat16)
                s = lax.dot_general(
                    qh, kk, (((1,), (1,)), ((), ())),
                    preferred_element_type=jnp.float32)
                s = s + bias[...]
                m = jnp.max(s, axis=-1, keepdims=True)
                e = jnp.exp(s - m)
                den = jnp.sum(e, axis=-1, keepdims=True)
                p = (e / den).astype(jnp.bfloat16)
                cv.wait()
                vv = vtile[...].astype(jnp.bfloat16)
                ctx = jnp.dot(p, vv,
                              preferred_element_type=jnp.float32)
                ctx = ctx.astype(jnp.bfloat16)
                woh = wo_ref[pl.ds(h * DH, DH), :]
                contrib[...] += jnp.dot(ctx, woh,
                                        preferred_element_type=jnp.float32)
                return carry

            lax.fori_loop(0, HQ, head_body, 0)

        buf[0, 0] = xb_ref[...]
        compute_contribution(my, xb_ref[...])
        buf[0, 1] = contrib[...].astype(jnp.bfloat16)

        for s in range(1, N_DEV + 1):
            src_slot = (s - 1) % 2
            dst_slot = s % 2
            if s >= 2:
                pl.semaphore_wait(credit_sem, 1)
            rdma = pltpu.make_async_remote_copy(
                src_ref=buf.at[src_slot],
                dst_ref=buf.at[dst_slot],
                send_sem=send_sems.at[src_slot],
                recv_sem=recv_sems.at[dst_slot],
                device_id=(right,),
                device_id_type=pl.DeviceIdType.MESH,
            )
            rdma.start()
            rdma.wait()
            if s <= N_DEV - 1:
                pl.semaphore_signal(credit_sem, inc=1, device_id=(left,),
                                    device_id_type=pl.DeviceIdType.MESH)
                b = lax.rem(my - s + N_DEV, N_DEV)
                compute_contribution(b, buf[dst_slot, 0])
                buf[dst_slot, 1] = (
                    buf[dst_slot, 1].astype(jnp.float32) + contrib[...]
                ).astype(jnp.bfloat16)
            else:
                out_ref[0] = buf[dst_slot, 1].astype(jnp.float32)

    out = pl.pallas_call(
        body,
        out_shape=jax.ShapeDtypeStruct((1, SQ, D), jnp.float32),
        in_specs=[
            pl.BlockSpec(memory_space=pltpu.VMEM),
            pl.BlockSpec(memory_space=pltpu.VMEM),
            pl.BlockSpec(memory_space=pltpu.VMEM),
            pl.BlockSpec(memory_space=pl.ANY),
            pl.BlockSpec(memory_space=pl.ANY),
        ],
        out_specs=pl.BlockSpec(memory_space=pltpu.VMEM),
        scratch_shapes=[
            pltpu.VMEM((2, 2, SQ, D), jnp.bfloat16),
            pltpu.VMEM((SQ, D), jnp.bfloat16),
            pltpu.VMEM((SQ, D), jnp.float32),
            pltpu.VMEM((SQ, SKV), jnp.bfloat16),
            pltpu.VMEM((SKV, DH), jnp.float32),
            pltpu.VMEM((SKV, DH), jnp.float32),
            pltpu.SemaphoreType.DMA((2,)),
            pltpu.SemaphoreType.DMA((2,)),
            pltpu.SemaphoreType.DMA((2,)),
            pltpu.SemaphoreType.REGULAR,
        ],
        compiler_params=pltpu.CompilerParams(collective_id=0),
    )(xb, wq, wo, K_ext, V_ext)
    return out


# device time: 297493 ns/iter; 2.4713x vs baseline; 2.4713x over previous
import jax
import jax.numpy as jnp
from jax import lax
from jax.experimental import pallas as pl
from jax.experimental.pallas import tpu as pltpu

N_DEV = 8
SQ = 1024
SKV = 1024
D = 1024
HQ = 8
HH = HQ // 2
DH = 128
HALF = HH * DH
SCALE = 0.08838834764831843
MESH = pl.DeviceIdType.MESH


def kernel(x, Wq, K_ext, V_ext, Wo):
    xb = x[0].astype(jnp.bfloat16)
    wq = Wq.astype(jnp.bfloat16)
    wo = Wo.astype(jnp.bfloat16)

    def body(xb_ref, wq_ref, wo_ref, kext_ref, vext_ref, out_ref,
             wq_cw, wo_cw, wq_ccw, wo_ccw, qbuf, acc, bias, kt, vt,
             r_sems, s_sems, kv_sems):
        my = lax.axis_index("i")
        left = lax.rem(my + N_DEV - 1, N_DEV)
        right = lax.rem(my + 1, N_DEV)

        barrier = pltpu.get_barrier_semaphore()
        for nbr in (left, right):
            pl.semaphore_signal(barrier, inc=1, device_id=(nbr,),
                                device_id_type=MESH)
        pl.semaphore_wait(barrier, 2)

        sent = []

        def isend(src_ref, dst_ref, row, hop, dev):
            d = pltpu.make_async_remote_copy(
                src_ref=src_ref, dst_ref=dst_ref,
                send_sem=s_sems.at[row, hop - 1],
                recv_sem=r_sems.at[row, hop - 1],
                device_id=(dev,), device_id_type=MESH)
            d.start()
            sent.append(d)

        def wait_recv(dst_ref, row, hop):
            d = pltpu.make_async_remote_copy(
                src_ref=dst_ref, dst_ref=dst_ref,
                send_sem=s_sems.at[row, hop - 1],
                recv_sem=r_sems.at[row, hop - 1],
                device_id=(left,), device_id_type=MESH)
            d.wait_recv()

        isend(wq_ref.at[:, pl.ds(0, HALF)], wq_cw.at[0], 0, 1, right)
        isend(wo_ref.at[pl.ds(0, HALF), :], wo_cw.at[0], 1, 1, right)
        isend(wq_ref.at[:, pl.ds(HALF, HALF)], wq_ccw.at[0], 2, 1, left)
        isend(wo_ref.at[pl.ds(HALF, HALF), :], wo_ccw.at[0], 3, 1, left)

        rblk = lax.broadcasted_iota(jnp.int32, (SQ, SKV), 0) // 64
        cblk = lax.broadcasted_iota(jnp.int32, (SQ, SKV), 1) // 64
        bias[...] = jnp.where(cblk <= rblk, 0.0, -1e9).astype(jnp.bfloat16)

        def compute_step(g1, g2, wqs_cw, wos_cw, wqs_ccw, wos_ccw):
            gh1 = g1 * HQ
            gh2 = g2 * HQ + HH
            for dirn, gh0 in ((0, gh1), (1, gh2)):
                for hp in range(HH):
                    pltpu.make_async_copy(
                        kext_ref.at[my, :, gh0 + hp, :], kt.at[dirn, hp],
                        kv_sems.at[dirn, 0, hp]).start()
                    pltpu.make_async_copy(
                        vext_ref.at[my, :, gh0 + hp, :], vt.at[dirn, hp],
                        kv_sems.at[dirn, 1, hp]).start()
            q1 = jnp.dot(xb_ref[...], wqs_cw[...],
                         preferred_element_type=jnp.float32)
            qbuf[:, pl.ds(0, HALF)] = (q1 * SCALE).astype(jnp.bfloat16)
            q2 = jnp.dot(xb_ref[...], wqs_ccw[...],
                         preferred_element_type=jnp.float32)
            qbuf[:, pl.ds(HALF, HALF)] = (q2 * SCALE).astype(jnp.bfloat16)

            for dirn, gh0, wos in ((0, gh1, wos_cw), (1, gh2, wos_ccw)):
                def head_body(h, carry, dirn=dirn, gh0=gh0, wos=wos):
                    gh = gh0 + h
                    pltpu.make_async_copy(
                        kext_ref.at[my, :, gh, :], kt.at[dirn, h],
                        kv_sems.at[dirn, 0, h]).wait()
                    qh = qbuf[:, pl.ds(dirn * HALF + h * DH, DH)]
                    kk = kt[dirn, h].astype(jnp.bfloat16)
                    s_ = lax.dot_general(
                        qh, kk, (((1,), (1,)), ((), ())),
                        preferred_element_type=jnp.float32)
                    s_ = s_ + bias[...]
                    m = jnp.max(s_, axis=-1, keepdims=True)
                    e = jnp.exp(s_ - m)
                    den = jnp.sum(e, axis=-1, keepdims=True)
                    p = (e / den).astype(jnp.bfloat16)
                    pltpu.make_async_copy(
                        vext_ref.at[my, :, gh, :], vt.at[dirn, h],
                        kv_sems.at[dirn, 1, h]).wait()
                    vv = vt[dirn, h].astype(jnp.bfloat16)
                    ctx = jnp.dot(p, vv, preferred_element_type=jnp.float32)
                    ctx = ctx.astype(jnp.bfloat16)
                    woh = wos[pl.ds(h * DH, DH), :]
                    acc[...] += jnp.dot(ctx, woh,
                                        preferred_element_type=jnp.float32)
                    return carry
                lax.fori_loop(0, HH, head_body, 0)

        acc[...] = jnp.zeros((SQ, D), jnp.float32)
        compute_step(my, my,
                     wq_ref.at[:, pl.ds(0, HALF)],
                     wo_ref.at[pl.ds(0, HALF), :],
                     wq_ref.at[:, pl.ds(HALF, HALF)],
                     wo_ref.at[pl.ds(HALF, HALF), :])

        for s in range(1, N_DEV):
            wait_recv(wq_cw.at[s - 1], 0, s)
            wait_recv(wo_cw.at[s - 1], 1, s)
            wait_recv(wq_ccw.at[s - 1], 2, s)
            wait_recv(wo_ccw.at[s - 1], 3, s)
            if s <= N_DEV - 2:
                isend(wq_cw.at[s - 1], wq_cw.at[s], 0, s + 1, right)
                isend(wo_cw.at[s - 1], wo_cw.at[s], 1, s + 1, right)
                isend(wq_ccw.at[s - 1], wq_ccw.at[s], 2, s + 1, left)
                isend(wo_ccw.at[s - 1], wo_ccw.at[s], 3, s + 1, left)
            g1 = lax.rem(my - s + N_DEV, N_DEV)
            g2 = lax.rem(my + s, N_DEV)
            compute_step(g1, g2,
                         wq_cw.at[s - 1], wo_cw.at[s - 1],
                         wq_ccw.at[s - 1], wo_ccw.at[s - 1])

        out_ref[0] = acc[...]
        for d in sent:
            d.wait_send()

    out = pl.pallas_call(
        body,
        out_shape=jax.ShapeDtypeStruct((1, SQ, D), jnp.float32),
        in_specs=[
            pl.BlockSpec(memory_space=pltpu.VMEM),
            pl.BlockSpec(memory_space=pltpu.VMEM),
            pl.BlockSpec(memory_space=pltpu.VMEM),
            pl.BlockSpec(memory_space=pl.ANY),
            pl.BlockSpec(memory_space=pl.ANY),
        ],
        out_specs=pl.BlockSpec(memory_space=pltpu.VMEM),
        scratch_shapes=[
            pltpu.VMEM((N_DEV - 1, D, HALF), jnp.bfloat16),
            pltpu.VMEM((N_DEV - 1, HALF, D), jnp.bfloat16),
            pltpu.VMEM((N_DEV - 1, D, HALF), jnp.bfloat16),
            pltpu.VMEM((N_DEV - 1, HALF, D), jnp.bfloat16),
            pltpu.VMEM((SQ, D), jnp.bfloat16),
            pltpu.VMEM((SQ, D), jnp.float32),
            pltpu.VMEM((SQ, SKV), jnp.bfloat16),
            pltpu.VMEM((2, HH, SKV, DH), jnp.float32),
            pltpu.VMEM((2, HH, SKV, DH), jnp.float32),
            pltpu.SemaphoreType.DMA((4, N_DEV - 1)),
            pltpu.SemaphoreType.DMA((4, N_DEV - 1)),
            pltpu.SemaphoreType.DMA((2, 2, HH)),
        ],
        compiler_params=pltpu.CompilerParams(
            collective_id=0, vmem_limit_bytes=100 * 1024 * 1024),
    )(xb, wq, wo, K_ext, V_ext)
    return out


# device time: 214930 ns/iter; 3.4206x vs baseline; 1.3841x over previous
import jax
import jax.numpy as jnp
from jax import lax
from jax.experimental import pallas as pl
from jax.experimental.pallas import tpu as pltpu

N_DEV = 8
SQ = 1024
SKV = 1024
D = 1024
HQ = 8
HH = HQ // 2
DH = 128
HALF = HH * DH
SCALE = 0.08838834764831843
MESH = pl.DeviceIdType.MESH


def kernel(x, Wq, K_ext, V_ext, Wo):
    xb = x[0].astype(jnp.bfloat16)
    wq = Wq.astype(jnp.bfloat16)
    wo = Wo.astype(jnp.bfloat16)

    def body(xb_ref, wq_ref, wo_ref, kext_ref, vext_ref, out_ref,
             wq_cw, wo_cw, wq_ccw, wo_ccw, qbuf, acc, bias, ctxbuf, kt, vt,
             r_sems, s_sems, kv_sems):
        my = lax.axis_index("i")
        left = lax.rem(my + N_DEV - 1, N_DEV)
        right = lax.rem(my + 1, N_DEV)

        barrier = pltpu.get_barrier_semaphore()
        for nbr in (left, right):
            pl.semaphore_signal(barrier, inc=1, device_id=(nbr,),
                                device_id_type=MESH)
        pl.semaphore_wait(barrier, 2)

        sent = []

        def isend(src_ref, dst_ref, row, hop, dev):
            d = pltpu.make_async_remote_copy(
                src_ref=src_ref, dst_ref=dst_ref,
                send_sem=s_sems.at[row, hop - 1],
                recv_sem=r_sems.at[row, hop - 1],
                device_id=(dev,), device_id_type=MESH)
            d.start()
            sent.append(d)

        def wait_recv(dst_ref, row, hop):
            d = pltpu.make_async_remote_copy(
                src_ref=dst_ref, dst_ref=dst_ref,
                send_sem=s_sems.at[row, hop - 1],
                recv_sem=r_sems.at[row, hop - 1],
                device_id=(left,), device_id_type=MESH)
            d.wait_recv()

        isend(wq_ref.at[:, pl.ds(0, HALF)], wq_cw.at[0], 0, 1, right)
        isend(wo_ref.at[pl.ds(0, HALF), :], wo_cw.at[0], 1, 1, right)
        isend(wq_ref.at[:, pl.ds(HALF, HALF)], wq_ccw.at[0], 2, 1, left)
        isend(wo_ref.at[pl.ds(HALF, HALF), :], wo_ccw.at[0], 3, 1, left)

        rblk = lax.broadcasted_iota(jnp.int32, (SQ, SKV), 0) // 64
        cblk = lax.broadcasted_iota(jnp.int32, (SQ, SKV), 1) // 64
        bias[...] = jnp.where(cblk <= rblk, 0.0, -1e9).astype(jnp.bfloat16)

        def compute_step(g1, g2, wqs_cw, wos_cw, wqs_ccw, wos_ccw):
            gh1 = g1 * HQ
            gh2 = g2 * HQ + HH
            for dirn, gh0 in ((0, gh1), (1, gh2)):
                for hp in range(HH):
                    pltpu.make_async_copy(
                        kext_ref.at[my, :, gh0 + hp, :], kt.at[dirn, hp],
                        kv_sems.at[dirn, 0, hp]).start()
                    pltpu.make_async_copy(
                        vext_ref.at[my, :, gh0 + hp, :], vt.at[dirn, hp],
                        kv_sems.at[dirn, 1, hp]).start()
            q1 = jnp.dot(xb_ref[...], wqs_cw[...],
                         preferred_element_type=jnp.float32)
            qbuf[:, pl.ds(0, HALF)] = (q1 * SCALE).astype(jnp.bfloat16)
            q2 = jnp.dot(xb_ref[...], wqs_ccw[...],
                         preferred_element_type=jnp.float32)
            qbuf[:, pl.ds(HALF, HALF)] = (q2 * SCALE).astype(jnp.bfloat16)

            for dirn, gh0, wos in ((0, gh1, wos_cw), (1, gh2, wos_ccw)):
                def head_body(h, carry, dirn=dirn, gh0=gh0):
                    gh = gh0 + h
                    pltpu.make_async_copy(
                        kext_ref.at[my, :, gh, :], kt.at[dirn, h],
                        kv_sems.at[dirn, 0, h]).wait()
                    pltpu.make_async_copy(
                        vext_ref.at[my, :, gh, :], vt.at[dirn, h],
                        kv_sems.at[dirn, 1, h]).wait()
                    for rb in range(2):
                        r0 = rb * 512
                        kvlen = r0 + 512
                        qh = qbuf[pl.ds(r0, 512),
                                  pl.ds(dirn * HALF + h * DH, DH)]
                        kk = kt[dirn, h, pl.ds(0, kvlen), :]
                        kk = kk.astype(jnp.bfloat16)
                        s_ = lax.dot_general(
                            qh, kk, (((1,), (1,)), ((), ())),
                            preferred_element_type=jnp.float32)
                        e = jnp.exp(s_ + bias[pl.ds(r0, 512), pl.ds(0, kvlen)])
                        den = jnp.sum(e, axis=-1, keepdims=True)
                        p = (e * (1.0 / den)).astype(jnp.bfloat16)
                        vv = vt[dirn, h, pl.ds(0, kvlen), :]
                        vv = vv.astype(jnp.bfloat16)
                        ctx = jnp.dot(p, vv,
                                      preferred_element_type=jnp.float32)
                        ctxbuf[pl.ds(r0, 512), pl.ds(h * DH, DH)] = (
                            ctx.astype(jnp.bfloat16))
                    return carry
                lax.fori_loop(0, HH, head_body, 0)
                acc[...] += jnp.dot(ctxbuf[...], wos[...],
                                    preferred_element_type=jnp.float32)

        acc[...] = jnp.zeros((SQ, D), jnp.float32)
        compute_step(my, my,
                     wq_ref.at[:, pl.ds(0, HALF)],
                     wo_ref.at[pl.ds(0, HALF), :],
                     wq_ref.at[:, pl.ds(HALF, HALF)],
                     wo_ref.at[pl.ds(HALF, HALF), :])

        for s in range(1, N_DEV):
            wait_recv(wq_cw.at[s - 1], 0, s)
            wait_recv(wo_cw.at[s - 1], 1, s)
            wait_recv(wq_ccw.at[s - 1], 2, s)
            wait_recv(wo_ccw.at[s - 1], 3, s)
            if s <= N_DEV - 2:
                isend(wq_cw.at[s - 1], wq_cw.at[s], 0, s + 1, right)
                isend(wo_cw.at[s - 1], wo_cw.at[s], 1, s + 1, right)
                isend(wq_ccw.at[s - 1], wq_ccw.at[s], 2, s + 1, left)
                isend(wo_ccw.at[s - 1], wo_ccw.at[s], 3, s + 1, left)
            g1 = lax.rem(my - s + N_DEV, N_DEV)
            g2 = lax.rem(my + s, N_DEV)
            compute_step(g1, g2,
                         wq_cw.at[s - 1], wo_cw.at[s - 1],
                         wq_ccw.at[s - 1], wo_ccw.at[s - 1])

        out_ref[0] = acc[...]
        for d in sent:
            d.wait_send()

    out = pl.pallas_call(
        body,
        out_shape=jax.ShapeDtypeStruct((1, SQ, D), jnp.float32),
        in_specs=[
            pl.BlockSpec(memory_space=pltpu.VMEM),
            pl.BlockSpec(memory_space=pltpu.VMEM),
            pl.BlockSpec(memory_space=pltpu.VMEM),
            pl.BlockSpec(memory_space=pl.ANY),
            pl.BlockSpec(memory_space=pl.ANY),
        ],
        out_specs=pl.BlockSpec(memory_space=pltpu.VMEM),
        scratch_shapes=[
            pltpu.VMEM((N_DEV - 1, D, HALF), jnp.bfloat16),
            pltpu.VMEM((N_DEV - 1, HALF, D), jnp.bfloat16),
            pltpu.VMEM((N_DEV - 1, D, HALF), jnp.bfloat16),
            pltpu.VMEM((N_DEV - 1, HALF, D), jnp.bfloat16),
            pltpu.VMEM((SQ, D), jnp.bfloat16),
            pltpu.VMEM((SQ, D), jnp.float32),
            pltpu.VMEM((SQ, SKV), jnp.bfloat16),
            pltpu.VMEM((SQ, HALF), jnp.bfloat16),
            pltpu.VMEM((2, HH, SKV, DH), jnp.float32),
            pltpu.VMEM((2, HH, SKV, DH), jnp.float32),
            pltpu.SemaphoreType.DMA((4, N_DEV - 1)),
            pltpu.SemaphoreType.DMA((4, N_DEV - 1)),
            pltpu.SemaphoreType.DMA((2, 2, HH)),
        ],
        compiler_params=pltpu.CompilerParams(
            collective_id=0, vmem_limit_bytes=100 * 1024 * 1024),
    )(xb, wq, wo, K_ext, V_ext)
    return out


# device time: 203753 ns/iter; 3.6083x vs baseline; 1.0549x over previous
import jax
import jax.numpy as jnp
from jax import lax
from jax.experimental import pallas as pl
from jax.experimental.pallas import tpu as pltpu

N_DEV = 8
SQ = 1024
SKV = 1024
D = 1024
HQ = 8
HH = HQ // 2
DH = 128
HALF = HH * DH
SCALE = 0.08838834764831843
MESH = pl.DeviceIdType.MESH


def kernel(x, Wq, K_ext, V_ext, Wo):
    xb = x[0].astype(jnp.bfloat16)
    wq = Wq.astype(jnp.bfloat16)
    wo = Wo.astype(jnp.bfloat16)

    def body(xb_ref, wq_ref, wo_ref, kext_ref, vext_ref, out_ref,
             wq_cw, wo_cw, wq_ccw, wo_ccw, qbuf, acc, bias, ctxbuf, kt, vt,
             r_sems, s_sems, kv_sems):
        my = lax.axis_index("i")
        left = lax.rem(my + N_DEV - 1, N_DEV)
        right = lax.rem(my + 1, N_DEV)

        barrier = pltpu.get_barrier_semaphore()
        for nbr in (left, right):
            pl.semaphore_signal(barrier, inc=1, device_id=(nbr,),
                                device_id_type=MESH)
        pl.semaphore_wait(barrier, 2)

        sent = []

        def isend(src_ref, dst_ref, row, hop, dev):
            d = pltpu.make_async_remote_copy(
                src_ref=src_ref, dst_ref=dst_ref,
                send_sem=s_sems.at[row, hop - 1],
                recv_sem=r_sems.at[row, hop - 1],
                device_id=(dev,), device_id_type=MESH)
            d.start()
            sent.append(d)

        def wait_recv(dst_ref, row, hop):
            d = pltpu.make_async_remote_copy(
                src_ref=dst_ref, dst_ref=dst_ref,
                send_sem=s_sems.at[row, hop - 1],
                recv_sem=r_sems.at[row, hop - 1],
                device_id=(left,), device_id_type=MESH)
            d.wait_recv()

        isend(wq_ref.at[:, pl.ds(0, HALF)], wq_cw.at[0], 0, 1, right)
        isend(wo_ref.at[pl.ds(0, HALF), :], wo_cw.at[0], 1, 1, right)
        isend(wq_ref.at[:, pl.ds(HALF, HALF)], wq_ccw.at[0], 2, 1, left)
        isend(wo_ref.at[pl.ds(HALF, HALF), :], wo_ccw.at[0], 3, 1, left)

        rows = lax.broadcasted_iota(jnp.int32, (SQ, 256), 0)
        cols = (rows // 256) * 256 + lax.broadcasted_iota(
            jnp.int32, (SQ, 256), 1)
        bias[...] = jnp.where(cols // 64 <= rows // 64, 0.0,
                              -1e9).astype(jnp.bfloat16)

        def compute_step(g1, g2, wqs_cw, wos_cw, wqs_ccw, wos_ccw):
            gh1 = g1 * HQ
            gh2 = g2 * HQ + HH
            for dirn, gh0 in ((0, gh1), (1, gh2)):
                for hp in range(HH):
                    pltpu.make_async_copy(
                        kext_ref.at[my, :, gh0 + hp, :], kt.at[dirn, hp],
                        kv_sems.at[dirn, 0, hp]).start()
                    pltpu.make_async_copy(
                        vext_ref.at[my, :, gh0 + hp, :], vt.at[dirn, hp],
                        kv_sems.at[dirn, 1, hp]).start()
            q1 = jnp.dot(xb_ref[...], wqs_cw[...],
                         preferred_element_type=jnp.float32)
            qbuf[:, pl.ds(0, HALF)] = (q1 * SCALE).astype(jnp.bfloat16)
            q2 = jnp.dot(xb_ref[...], wqs_ccw[...],
                         preferred_element_type=jnp.float32)
            qbuf[:, pl.ds(HALF, HALF)] = (q2 * SCALE).astype(jnp.bfloat16)

            for dirn, gh0, wos in ((0, gh1, wos_cw), (1, gh2, wos_ccw)):
                def head_body(h, carry, dirn=dirn, gh0=gh0):
                    gh = gh0 + h
                    pltpu.make_async_copy(
                        kext_ref.at[my, :, gh, :], kt.at[dirn, h],
                        kv_sems.at[dirn, 0, h]).wait()
                    pltpu.make_async_copy(
                        vext_ref.at[my, :, gh, :], vt.at[dirn, h],
                        kv_sems.at[dirn, 1, h]).wait()
                    kk = kt[dirn, h].astype(jnp.bfloat16)
                    vv = vt[dirn, h].astype(jnp.bfloat16)
                    for rb in range(4):
                        r0 = rb * 256
                        kvlen = r0 + 256
                        qh = qbuf[pl.ds(r0, 256),
                                  pl.ds(dirn * HALF + h * DH, DH)]
                        s_ = lax.dot_general(
                            qh, kk[:kvlen], (((1,), (1,)), ((), ())),
                            preferred_element_type=jnp.float32)
                        e_tail = jnp.exp(s_[:, r0:] + bias[pl.ds(r0, 256), :])
                        den = jnp.sum(e_tail, axis=-1, keepdims=True)
                        if r0 > 0:
                            e_head = jnp.exp(s_[:, :r0])
                            den = den + jnp.sum(e_head, axis=-1,
                                                keepdims=True)
                        rcp = 1.0 / den
                        p_tail = (e_tail * rcp).astype(jnp.bfloat16)
                        ctx = jnp.dot(p_tail, vv[r0:kvlen],
                                      preferred_element_type=jnp.float32)
                        if r0 > 0:
                            p_head = (e_head * rcp).astype(jnp.bfloat16)
                            ctx = ctx + jnp.dot(
                                p_head, vv[:r0],
                                preferred_element_type=jnp.float32)
                        ctxbuf[pl.ds(r0, 256), pl.ds(h * DH, DH)] = (
                            ctx.astype(jnp.bfloat16))
                    return carry
                lax.fori_loop(0, HH, head_body, 0)
                acc[...] += jnp.dot(ctxbuf[...], wos[...],
                                    preferred_element_type=jnp.float32)

        acc[...] = jnp.zeros((SQ, D), jnp.float32)
        compute_step(my, my,
                     wq_ref.at[:, pl.ds(0, HALF)],
                     wo_ref.at[pl.ds(0, HALF), :],
                     wq_ref.at[:, pl.ds(HALF, HALF)],
                     wo_ref.at[pl.ds(HALF, HALF), :])

        for s in range(1, N_DEV):
            fwd = s <= N_DEV - 2
            wait_recv(wq_cw.at[s - 1], 0, s)
            if fwd:
                isend(wq_cw.at[s - 1], wq_cw.at[s], 0, s + 1, right)
            wait_recv(wq_ccw.at[s - 1], 2, s)
            if fwd:
                isend(wq_ccw.at[s - 1], wq_ccw.at[s], 2, s + 1, left)
            wait_recv(wo_cw.at[s - 1], 1, s)
            if fwd:
                isend(wo_cw.at[s - 1], wo_cw.at[s], 1, s + 1, right)
            wait_recv(wo_ccw.at[s - 1], 3, s)
            if fwd:
                isend(wo_ccw.at[s - 1], wo_ccw.at[s], 3, s + 1, left)
            g1 = lax.rem(my - s + N_DEV, N_DEV)
            g2 = lax.rem(my + s, N_DEV)
            compute_step(g1, g2,
                         wq_cw.at[s - 1], wo_cw.at[s - 1],
                         wq_ccw.at[s - 1], wo_ccw.at[s - 1])

        out_ref[0] = acc[...]
        for d in sent:
            d.wait_send()

    out = pl.pallas_call(
        body,
        out_shape=jax.ShapeDtypeStruct((1, SQ, D), jnp.float32),
        in_specs=[
            pl.BlockSpec(memory_space=pltpu.VMEM),
            pl.BlockSpec(memory_space=pltpu.VMEM),
            pl.BlockSpec(memory_space=pltpu.VMEM),
            pl.BlockSpec(memory_space=pl.ANY),
            pl.BlockSpec(memory_space=pl.ANY),
        ],
        out_specs=pl.BlockSpec(memory_space=pltpu.VMEM),
        scratch_shapes=[
            pltpu.VMEM((N_DEV - 1, D, HALF), jnp.bfloat16),
            pltpu.VMEM((N_DEV - 1, HALF, D), jnp.bfloat16),
            pltpu.VMEM((N_DEV - 1, D, HALF), jnp.bfloat16),
            pltpu.VMEM((N_DEV - 1, HALF, D), jnp.bfloat16),
            pltpu.VMEM((SQ, D), jnp.bfloat16),
            pltpu.VMEM((SQ, D), jnp.float32),
            pltpu.VMEM((SQ, 256), jnp.bfloat16),
            pltpu.VMEM((SQ, HALF), jnp.bfloat16),
            pltpu.VMEM((2, HH, SKV, DH), jnp.float32),
            pltpu.VMEM((2, HH, SKV, DH), jnp.float32),
            pltpu.SemaphoreType.DMA((4, N_DEV - 1)),
            pltpu.SemaphoreType.DMA((4, N_DEV - 1)),
            pltpu.SemaphoreType.DMA((2, 2, HH)),
        ],
        compiler_params=pltpu.CompilerParams(
            collective_id=0, vmem_limit_bytes=100 * 1024 * 1024),
    )(xb, wq, wo, K_ext, V_ext)
    return out


# device time: 196063 ns/iter; 3.7498x vs baseline; 1.0392x over previous
import jax
import jax.numpy as jnp
from jax import lax
from jax.experimental import pallas as pl
from jax.experimental.pallas import tpu as pltpu

N_DEV = 8
SQ = 1024
SKV = 1024
D = 1024
HQ = 8
HH = HQ // 2
DH = 128
HALF = HH * DH
SCALE = 0.08838834764831843
MESH = pl.DeviceIdType.MESH


def kernel(x, Wq, K_ext, V_ext, Wo):
    xb = x[0].astype(jnp.bfloat16)
    wq = Wq.astype(jnp.bfloat16)
    wo = Wo.astype(jnp.bfloat16)

    def body(xb_ref, wq_ref, wo_ref, kext_ref, vext_ref, out_ref,
             wq_cw, wo_cw, wq_ccw, wo_ccw, qbuf, acc, bias, ctxbuf, kt, vt,
             r_sems, s_sems, kv_sems):
        my = lax.axis_index("i")
        left = lax.rem(my + N_DEV - 1, N_DEV)
        right = lax.rem(my + 1, N_DEV)

        barrier = pltpu.get_barrier_semaphore()
        for nbr in (left, right):
            pl.semaphore_signal(barrier, inc=1, device_id=(nbr,),
                                device_id_type=MESH)
        pl.semaphore_wait(barrier, 2)

        sent = []

        def isend(src_ref, dst_ref, row, hop, dev):
            d = pltpu.make_async_remote_copy(
                src_ref=src_ref, dst_ref=dst_ref,
                send_sem=s_sems.at[row, hop - 1],
                recv_sem=r_sems.at[row, hop - 1],
                device_id=(dev,), device_id_type=MESH)
            d.start()
            sent.append(d)

        def wait_recv(dst_ref, row, hop):
            d = pltpu.make_async_remote_copy(
                src_ref=dst_ref, dst_ref=dst_ref,
                send_sem=s_sems.at[row, hop - 1],
                recv_sem=r_sems.at[row, hop - 1],
                device_id=(left,), device_id_type=MESH)
            d.wait_recv()

        isend(wq_ref.at[:, pl.ds(0, HALF)], wq_cw.at[0], 0, 1, right)
        isend(wo_ref.at[pl.ds(0, HALF), :], wo_cw.at[0], 1, 1, right)
        isend(wq_ref.at[:, pl.ds(HALF, HALF)], wq_ccw.at[0], 2, 1, left)
        isend(wo_ref.at[pl.ds(HALF, HALF), :], wo_ccw.at[0], 3, 1, left)

        rows = lax.broadcasted_iota(jnp.int32, (SQ, 256), 0)
        cols = (rows // 256) * 256 + lax.broadcasted_iota(
            jnp.int32, (SQ, 256), 1)
        bias[...] = jnp.where(cols // 64 <= rows // 64, 0.0,
                              -1e9).astype(jnp.bfloat16)

        def compute_step(g1, g2, wqs_cw, wos_cw, wqs_ccw, wos_ccw, hooks):
            h_wq_cw, h_wo_cw, h_wq_ccw, h_wo_ccw = hooks
            gh1 = g1 * HQ
            gh2 = g2 * HQ + HH
            for dirn, gh0 in ((0, gh1), (1, gh2)):
                for hp in range(HH):
                    pltpu.make_async_copy(
                        kext_ref.at[my, :, gh0 + hp, :], kt.at[dirn, hp],
                        kv_sems.at[dirn, 0, hp]).start()
                    pltpu.make_async_copy(
                        vext_ref.at[my, :, gh0 + hp, :], vt.at[dirn, hp],
                        kv_sems.at[dirn, 1, hp]).start()
            h_wq_cw()
            q1 = jnp.dot(xb_ref[...], wqs_cw[...],
                         preferred_element_type=jnp.float32)
            qbuf[:, pl.ds(0, HALF)] = (q1 * SCALE).astype(jnp.bfloat16)
            h_wq_ccw()
            q2 = jnp.dot(xb_ref[...], wqs_ccw[...],
                         preferred_element_type=jnp.float32)
            qbuf[:, pl.ds(HALF, HALF)] = (q2 * SCALE).astype(jnp.bfloat16)

            for dirn, gh0, wos, h_wo in ((0, gh1, wos_cw, h_wo_cw),
                                         (1, gh2, wos_ccw, h_wo_ccw)):
                def head_body(h, carry, dirn=dirn, gh0=gh0):
                    gh = gh0 + h
                    pltpu.make_async_copy(
                        kext_ref.at[my, :, gh, :], kt.at[dirn, h],
                        kv_sems.at[dirn, 0, h]).wait()
                    pltpu.make_async_copy(
                        vext_ref.at[my, :, gh, :], vt.at[dirn, h],
                        kv_sems.at[dirn, 1, h]).wait()
                    kk = kt[dirn, h].astype(jnp.bfloat16)
                    vv = vt[dirn, h].astype(jnp.bfloat16)
                    for rb in range(4):
                        r0 = rb * 256
                        kvlen = r0 + 256
                        qh = qbuf[pl.ds(r0, 256),
                                  pl.ds(dirn * HALF + h * DH, DH)]
                        s_ = lax.dot_general(
                            qh, kk[:kvlen], (((1,), (1,)), ((), ())),
                            preferred_element_type=jnp.float32)
                        e_tail = jnp.exp(s_[:, r0:] + bias[pl.ds(r0, 256), :])
                        den = jnp.sum(e_tail, axis=-1, keepdims=True)
                        if r0 > 0:
                            e_head = jnp.exp(s_[:, :r0])
                            den = den + jnp.sum(e_head, axis=-1,
                                                keepdims=True)
                        rcp = 1.0 / den
                        p_tail = (e_tail * rcp).astype(jnp.bfloat16)
                        ctx = jnp.dot(p_tail, vv[r0:kvlen],
                                      preferred_element_type=jnp.float32)
                        if r0 > 0:
                            p_head = (e_head * rcp).astype(jnp.bfloat16)
                            ctx = ctx + jnp.dot(
                                p_head, vv[:r0],
                                preferred_element_type=jnp.float32)
                        ctxbuf[pl.ds(r0, 256), pl.ds(h * DH, DH)] = (
                            ctx.astype(jnp.bfloat16))
                    return carry
                lax.fori_loop(0, HH, head_body, 0)
                h_wo()
                acc[...] += jnp.dot(ctxbuf[...], wos[...],
                                    preferred_element_type=jnp.float32)

        acc[...] = jnp.zeros((SQ, D), jnp.float32)
        noop = lambda: None
        compute_step(my, my,
                     wq_ref.at[:, pl.ds(0, HALF)],
                     wo_ref.at[pl.ds(0, HALF), :],
                     wq_ref.at[:, pl.ds(HALF, HALF)],
                     wo_ref.at[pl.ds(HALF, HALF), :],
                     (noop, noop, noop, noop))

        for s in range(1, N_DEV):
            fwd = s <= N_DEV - 2
            slots = (wq_cw, wo_cw, wq_ccw, wo_ccw)
            devs = (right, right, left, left)

            def mk_hook(row, s=s, fwd=fwd):
                buf_ = slots[row]
                def hook(row=row, buf_=buf_):
                    wait_recv(buf_.at[s - 1], row, s)
                    if fwd:
                        isend(buf_.at[s - 1], buf_.at[s], row, s + 1,
                              devs[row])
                return hook

            g1 = lax.rem(my - s + N_DEV, N_DEV)
            g2 = lax.rem(my + s, N_DEV)
            compute_step(g1, g2,
                         wq_cw.at[s - 1], wo_cw.at[s - 1],
                         wq_ccw.at[s - 1], wo_ccw.at[s - 1],
                         (mk_hook(0), mk_hook(1), mk_hook(2), mk_hook(3)))

        out_ref[0] = acc[...]
        for d in sent:
            d.wait_send()

    out = pl.pallas_call(
        body,
        out_shape=jax.ShapeDtypeStruct((1, SQ, D), jnp.float32),
        in_specs=[
            pl.BlockSpec(memory_space=pltpu.VMEM),
            pl.BlockSpec(memory_space=pltpu.VMEM),
            pl.BlockSpec(memory_space=pltpu.VMEM),
            pl.BlockSpec(memory_space=pl.ANY),
            pl.BlockSpec(memory_space=pl.ANY),
        ],
        out_specs=pl.BlockSpec(memory_space=pltpu.VMEM),
        scratch_shapes=[
            pltpu.VMEM((N_DEV - 1, D, HALF), jnp.bfloat16),
            pltpu.VMEM((N_DEV - 1, HALF, D), jnp.bfloat16),
            pltpu.VMEM((N_DEV - 1, D, HALF), jnp.bfloat16),
            pltpu.VMEM((N_DEV - 1, HALF, D), jnp.bfloat16),
            pltpu.VMEM((SQ, D), jnp.bfloat16),
            pltpu.VMEM((SQ, D), jnp.float32),
            pltpu.VMEM((SQ, 256), jnp.bfloat16),
            pltpu.VMEM((SQ, HALF), jnp.bfloat16),
            pltpu.VMEM((2, HH, SKV, DH), jnp.float32),
            pltpu.VMEM((2, HH, SKV, DH), jnp.float32),
            pltpu.SemaphoreType.DMA((4, N_DEV - 1)),
            pltpu.SemaphoreType.DMA((4, N_DEV - 1)),
            pltpu.SemaphoreType.DMA((2, 2, HH)),
        ],
        compiler_params=pltpu.CompilerParams(
            collective_id=0, vmem_limit_bytes=100 * 1024 * 1024),
    )(xb, wq, wo, K_ext, V_ext)
    return out
